# baseline (speedup 1.0000x reference)
import jax
import jax.numpy as jnp
from jax.experimental import pallas as pl
from jax.experimental.pallas import tpu as pltpu

QUANT_TEMP = 0.3
TILE = 1024
N_EXPERTS = 7


def _route(z, a, rtc):
    q = jnp.tanh(z / QUANT_TEMP)
    dott = jax.lax.dot_general(
        a, q, (((1,), (0,)), ((), ())),
        preferred_element_type=jnp.float32)
    hamming = (6.0 - dott) / 2.0
    l = -hamming / rtc
    iota = jax.lax.broadcasted_iota(jnp.int32, l.shape, 0)
    m1 = jnp.max(l, axis=0, keepdims=True)
    i1 = jnp.min(jnp.where(l == m1, iota, N_EXPERTS), axis=0, keepdims=True)
    masked = jnp.where(iota == i1, -jnp.inf, l)
    m2 = jnp.max(masked, axis=0, keepdims=True)
    i2 = jnp.min(jnp.where(masked == m2, iota, N_EXPERTS), axis=0,
                 keepdims=True)
    e2 = jnp.exp(m2 - m1)
    denom = 1.0 + e2
    ew = (jnp.where(iota == i1, 1.0 / denom, 0.0)
          + jnp.where(iota == i2, e2 / denom, 0.0))
    return q, ew


def _fused_body(xa_ref, xb_ref, wt_ref, a_ref, rtc_ref, q_ref, ew_ref):
    wt = wt_ref[...]
    a = a_ref[...]
    rtc = rtc_ref[...]
    for s, x_ref in enumerate((xa_ref, xb_ref)):
        z = jax.lax.dot_general(
            x_ref[0], wt, (((1,), (0,)), ((), ())),
            preferred_element_type=jnp.float32)
        q, ew = _route(z.T, a, rtc)
        q_ref[s] = q
        ew_ref[s] = ew


@jax.jit
def kernel(x, W, anchors, routing_temp):
    B, T, D = x.shape
    n = B * T
    h = n // 2
    xr = x.reshape(2, h, D)
    rtc = jnp.maximum(routing_temp, 0.1).reshape(1, 1)
    wt = W.T
    grid = (h // TILE,)
    q2, ew2 = pl.pallas_call(
        _fused_body,
        grid=grid,
        in_specs=[
            pl.BlockSpec((1, TILE, D), lambda i: (0, i, 0)),
            pl.BlockSpec((1, TILE, D), lambda i: (1, i, 0)),
            pl.BlockSpec((D, 6), lambda i: (0, 0)),
            pl.BlockSpec((N_EXPERTS, 6), lambda i: (0, 0)),
            pl.BlockSpec((1, 1), lambda i: (0, 0)),
        ],
        out_specs=[
            pl.BlockSpec((2, 6, TILE), lambda i: (0, 0, i)),
            pl.BlockSpec((2, N_EXPERTS, TILE), lambda i: (0, 0, i)),
        ],
        out_shape=[
            jax.ShapeDtypeStruct((2, 6, h), jnp.float32),
            jax.ShapeDtypeStruct((2, N_EXPERTS, h), jnp.float32),
        ],
        compiler_params=pltpu.CompilerParams(
            dimension_semantics=("parallel",)),
    )(xr, xr, wt, anchors, rtc)
    ew = ew2.transpose(0, 2, 1).reshape(B, T, N_EXPERTS)
    q = q2.transpose(0, 2, 1).reshape(B, T, 6)
    return ew, q


# speedup vs baseline: 1.0739x; 1.0739x over previous
import jax
import jax.numpy as jnp
from jax.experimental import pallas as pl
from jax.experimental.pallas import tpu as pltpu

QUANT_TEMP = 0.3
TILE = 1024
N_EXPERTS = 7


def _route(z, a, rtc):
    q = jnp.tanh(z / QUANT_TEMP)
    dott = jax.lax.dot_general(
        a, q, (((1,), (0,)), ((), ())),
        preferred_element_type=jnp.float32)
    hamming = (6.0 - dott) / 2.0
    l = -hamming / rtc
    iota = jax.lax.broadcasted_iota(jnp.int32, l.shape, 0)
    m1 = jnp.max(l, axis=0, keepdims=True)
    i1 = jnp.min(jnp.where(l == m1, iota, N_EXPERTS), axis=0, keepdims=True)
    masked = jnp.where(iota == i1, -jnp.inf, l)
    m2 = jnp.max(masked, axis=0, keepdims=True)
    i2 = jnp.min(jnp.where(masked == m2, iota, N_EXPERTS), axis=0,
                 keepdims=True)
    e2 = jnp.exp(m2 - m1)
    denom = 1.0 + e2
    ew = (jnp.where(iota == i1, 1.0 / denom, 0.0)
          + jnp.where(iota == i2, e2 / denom, 0.0))
    return q, ew


def _fused_body(x_ref, wt_ref, a_ref, rtc_ref, q_ref, ew_ref):
    xab = x_ref[...].reshape(2 * TILE, x_ref.shape[2])
    z = jax.lax.dot_general(
        xab, wt_ref[...], (((1,), (0,)), ((), ())),
        preferred_element_type=jnp.float32)
    q, ew = _route(z.T, a_ref[...], rtc_ref[...])
    q_ref[0] = q[:, :TILE]
    q_ref[1] = q[:, TILE:]
    ew_ref[0] = ew[:, :TILE]
    ew_ref[1] = ew[:, TILE:]


@jax.jit
def kernel(x, W, anchors, routing_temp):
    B, T, D = x.shape
    n = B * T
    h = n // 2
    xr = x.reshape(2, h, D)
    rtc = jnp.maximum(routing_temp, 0.1).reshape(1, 1)
    wt = W.T
    grid = (h // TILE,)
    q2, ew2 = pl.pallas_call(
        _fused_body,
        grid=grid,
        in_specs=[
            pl.BlockSpec((2, TILE, D), lambda i: (0, i, 0)),
            pl.BlockSpec((D, 6), lambda i: (0, 0)),
            pl.BlockSpec((N_EXPERTS, 6), lambda i: (0, 0)),
            pl.BlockSpec((1, 1), lambda i: (0, 0)),
        ],
        out_specs=[
            pl.BlockSpec((2, 6, TILE), lambda i: (0, 0, i)),
            pl.BlockSpec((2, N_EXPERTS, TILE), lambda i: (0, 0, i)),
        ],
        out_shape=[
            jax.ShapeDtypeStruct((2, 6, h), jnp.float32),
            jax.ShapeDtypeStruct((2, N_EXPERTS, h), jnp.float32),
        ],
        compiler_params=pltpu.CompilerParams(
            dimension_semantics=("parallel",)),
    )(xr, wt, anchors, rtc)
    ew = ew2.transpose(0, 2, 1).reshape(B, T, N_EXPERTS)
    q = q2.transpose(0, 2, 1).reshape(B, T, 6)
    return ew, q


# speedup vs baseline: 1.0826x; 1.0081x over previous
import jax
import jax.numpy as jnp
from jax.experimental import pallas as pl
from jax.experimental.pallas import tpu as pltpu

QUANT_TEMP = 0.3
TILE = 1024
N_EXPERTS = 7


def _route(z, a, rtc):
    q = jnp.tanh(z / QUANT_TEMP)
    dott = jax.lax.dot_general(
        a, q, (((1,), (0,)), ((), ())),
        preferred_element_type=jnp.float32)
    hamming = (6.0 - dott) / 2.0
    l = -hamming / rtc
    iota = jax.lax.broadcasted_iota(jnp.int32, l.shape, 0)
    m1 = jnp.max(l, axis=0, keepdims=True)
    i1 = jnp.min(jnp.where(l == m1, iota, N_EXPERTS), axis=0, keepdims=True)
    masked = jnp.where(iota == i1, -jnp.inf, l)
    m2 = jnp.max(masked, axis=0, keepdims=True)
    i2 = jnp.min(jnp.where(masked == m2, iota, N_EXPERTS), axis=0,
                 keepdims=True)
    e2 = jnp.exp(m2 - m1)
    denom = 1.0 + e2
    ew = (jnp.where(iota == i1, 1.0 / denom, 0.0)
          + jnp.where(iota == i2, e2 / denom, 0.0))
    return q, ew


def _fused_body(xa_ref, xb_ref, wt_ref, a_ref, rtc_ref, q_ref, ew_ref):
    wt = wt_ref[...]
    za = jax.lax.dot_general(
        xa_ref[0], wt, (((1,), (0,)), ((), ())),
        preferred_element_type=jnp.float32)
    zb = jax.lax.dot_general(
        xb_ref[0], wt, (((1,), (0,)), ((), ())),
        preferred_element_type=jnp.float32)
    zt = jnp.concatenate([za.T, zb.T], axis=1)
    q, ew = _route(zt, a_ref[...], rtc_ref[...])
    q_ref[0] = q[:, :TILE]
    q_ref[1] = q[:, TILE:]
    ew_ref[0] = ew[:, :TILE]
    ew_ref[1] = ew[:, TILE:]


@jax.jit
def kernel(x, W, anchors, routing_temp):
    B, T, D = x.shape
    n = B * T
    h = n // 2
    xr = x.reshape(2, h, D)
    rtc = jnp.maximum(routing_temp, 0.1).reshape(1, 1)
    wt = W.T
    grid = (h // TILE,)
    q2, ew2 = pl.pallas_call(
        _fused_body,
        grid=grid,
        in_specs=[
            pl.BlockSpec((1, TILE, D), lambda i: (0, i, 0)),
            pl.BlockSpec((1, TILE, D), lambda i: (1, i, 0)),
            pl.BlockSpec((D, 6), lambda i: (0, 0)),
            pl.BlockSpec((N_EXPERTS, 6), lambda i: (0, 0)),
            pl.BlockSpec((1, 1), lambda i: (0, 0)),
        ],
        out_specs=[
            pl.BlockSpec((2, 6, TILE), lambda i: (0, 0, i)),
            pl.BlockSpec((2, N_EXPERTS, TILE), lambda i: (0, 0, i)),
        ],
        out_shape=[
            jax.ShapeDtypeStruct((2, 6, h), jnp.float32),
            jax.ShapeDtypeStruct((2, N_EXPERTS, h), jnp.float32),
        ],
        compiler_params=pltpu.CompilerParams(
            dimension_semantics=("parallel",)),
    )(xr, xr, wt, anchors, rtc)
    ew = ew2.transpose(0, 2, 1).reshape(B, T, N_EXPERTS)
    q = q2.transpose(0, 2, 1).reshape(B, T, 6)
    return ew, q


# speedup vs baseline: 1.0941x; 1.0106x over previous
import jax
import jax.numpy as jnp
from jax.experimental import pallas as pl
from jax.experimental.pallas import tpu as pltpu

QUANT_TEMP = 0.3
TILE = 2048
N_EXPERTS = 7


def _fused_body(x_ref, wt_ref, a_ref, rtc_ref, q_ref, ew_ref):
    xt = x_ref[...]
    z = jax.lax.dot_general(
        xt, wt_ref[...], (((1,), (0,)), ((), ())),
        preferred_element_type=jnp.float32)
    zt = z.T
    qt = jnp.tanh(zt / QUANT_TEMP)
    q_ref[...] = qt
    dott = jax.lax.dot_general(
        a_ref[...], qt, (((1,), (0,)), ((), ())),
        preferred_element_type=jnp.float32)
    hamming = (6.0 - dott) / 2.0
    l = -hamming / rtc_ref[...]
    iota = jax.lax.broadcasted_iota(jnp.int32, l.shape, 0)
    m1 = jnp.max(l, axis=0, keepdims=True)
    i1 = jnp.min(jnp.where(l == m1, iota, N_EXPERTS), axis=0, keepdims=True)
    masked = jnp.where(iota == i1, -jnp.inf, l)
    m2 = jnp.max(masked, axis=0, keepdims=True)
    i2 = jnp.min(jnp.where(masked == m2, iota, N_EXPERTS), axis=0,
                 keepdims=True)
    e2 = jnp.exp(m2 - m1)
    denom = 1.0 + e2
    ew_ref[...] = (jnp.where(iota == i1, 1.0 / denom, 0.0)
                   + jnp.where(iota == i2, e2 / denom, 0.0))


@jax.jit
def kernel(x, W, anchors, routing_temp):
    B, T, D = x.shape
    n = B * T
    xf = x.reshape(n, D)
    rtc = jnp.maximum(routing_temp, 0.1).reshape(1, 1)
    wt = W.T
    grid = (n // TILE,)
    q, ew = pl.pallas_call(
        _fused_body,
        grid=grid,
        in_specs=[
            pl.BlockSpec((TILE, D), lambda i: (i, 0)),
            pl.BlockSpec((D, 6), lambda i: (0, 0)),
            pl.BlockSpec((N_EXPERTS, 6), lambda i: (0, 0)),
            pl.BlockSpec((1, 1), lambda i: (0, 0)),
        ],
        out_specs=[
            pl.BlockSpec((6, TILE), lambda i: (0, i)),
            pl.BlockSpec((N_EXPERTS, TILE), lambda i: (0, i)),
        ],
        out_shape=[
            jax.ShapeDtypeStruct((6, n), jnp.float32),
            jax.ShapeDtypeStruct((N_EXPERTS, n), jnp.float32),
        ],
        compiler_params=pltpu.CompilerParams(
            dimension_semantics=("parallel",)),
    )(xf, wt, anchors, rtc)
    return ew.T.reshape(B, T, N_EXPERTS), q.T.reshape(B, T, 6)
